# baseline (speedup 1.0000x reference)
import functools

import jax
import jax.numpy as jnp
from jax import lax
from jax.experimental import pallas as pl
from jax.experimental.pallas import tpu as pltpu
from jax.experimental.pallas import tpu_sc as plsc


def _make_sc_kernel(B, D, N):
    info = plsc.get_sparse_core_info()
    NC, NS, L = info.num_cores, info.num_subcores, info.num_lanes
    NW = NC * NS
    BW = B // NW
    PH = BW // 2
    CH = 128
    W = 2 * D

    mesh = plsc.VectorSubcoreMesh(core_axis_name="c", subcore_axis_name="s")

    @functools.partial(
        pl.kernel,
        mesh=mesh,
        compiler_params=pltpu.CompilerParams(
            needs_layout_passes=False,
        ),
        out_type=jax.ShapeDtypeStruct((B,), jnp.float32),
        scratch_types=[
            pltpu.VMEM((BW,), jnp.int32),
            pltpu.VMEM((BW,), jnp.int32),
            pltpu.VMEM((BW,), jnp.int32),
            pltpu.VMEM((BW,), jnp.int32),
            pltpu.VMEM((PH, W), jnp.float32),
            pltpu.VMEM((PH, W), jnp.float32),
            pltpu.VMEM((BW,), jnp.float32),
            pltpu.VMEM((BW,), jnp.float32),
            pltpu.VMEM((BW,), jnp.float32),
            pltpu.SemaphoreType.DMA,
        ],
    )
    def k(item_hbm, ctx_hbm, emb_hbm, bias_hbm, out_hbm,
          iidx, cidx, irow2, crow2, irows, crows, ibv, cbv, ov, sem):
        wid = lax.axis_index("s") * NC + lax.axis_index("c")
        base = wid * BW
        pltpu.sync_copy(item_hbm.at[pl.ds(base, BW)], iidx)
        pltpu.sync_copy(ctx_hbm.at[pl.ds(base, BW)], cidx)

        def shift_group(g, carry):
            sl = pl.ds(g * L, L)
            irow2[sl] = lax.shift_right_logical(iidx[sl], 1)
            crow2[sl] = lax.shift_right_logical(cidx[sl], 1)
            return carry

        lax.fori_loop(0, BW // L, shift_group, 0)

        bias_copies = []
        for j in range(BW // CH):
            sl = pl.ds(j * CH, CH)
            bias_copies.append(
                pltpu.async_copy(bias_hbm.at[iidx.at[sl]], ibv.at[sl], sem))
            bias_copies.append(
                pltpu.async_copy(bias_hbm.at[cidx.at[sl]], cbv.at[sl], sem))

        def gather_phase(ph0):
            copies = []
            for j in range(PH // CH):
                isl = pl.ds(ph0 + j * CH, CH)
                dsl = pl.ds(j * CH, CH)
                copies.append(pltpu.async_copy(
                    emb_hbm.at[irow2.at[isl]], irows.at[dsl], sem))
                copies.append(pltpu.async_copy(
                    emb_hbm.at[crow2.at[isl]], crows.at[dsl], sem))
            return copies

        lane_ids = lax.iota(jnp.int32, L)

        def compute_phase(ph0):
            def group(g, carry):
                row0 = g * L
                iv = iidx[pl.ds(ph0 + row0, L)]
                cv = cidx[pl.ds(ph0 + row0, L)]
                sums = jnp.zeros((L,), jnp.float32)
                for r in range(L):
                    row = row0 + r
                    ih = (iv[r] & 1) * D
                    chh = (cv[r] & 1) * D
                    acc = (irows[row, pl.ds(ih, L)]
                           * crows[row, pl.ds(chh, L)])
                    for c in range(1, D // L):
                        acc = acc + (irows[row, pl.ds(ih + c * L, L)]
                                     * crows[row, pl.ds(chh + c * L, L)])
                    sums = jnp.where(lane_ids == r, jnp.sum(acc), sums)
                osl = pl.ds(ph0 + row0, L)
                ov[osl] = sums + ibv[osl] + cbv[osl]
                return carry
            lax.fori_loop(0, PH // L, group, 0)

        for c in bias_copies:
            c.wait()
        for ph in range(2):
            copies = gather_phase(ph * PH)
            for c in copies:
                c.wait()
            compute_phase(ph * PH)

        pltpu.sync_copy(ov, out_hbm.at[pl.ds(base, BW)])

    return k


def kernel(item_ids, context_ids, emb_table, bias_table):
    B = item_ids.shape[0]
    N, D = emb_table.shape
    emb_packed = emb_table.reshape(N // 2, 2 * D)
    bias_flat = bias_table.reshape(-1)
    k = _make_sc_kernel(B, D, N)
    return k(item_ids.astype(jnp.int32), context_ids.astype(jnp.int32),
             emb_packed, bias_flat)


# speedup vs baseline: 1.3936x; 1.3936x over previous
import functools

import jax
import jax.numpy as jnp
from jax import lax
from jax.experimental import pallas as pl
from jax.experimental.pallas import tpu as pltpu
from jax.experimental.pallas import tpu_sc as plsc


def _make_sc_kernel(B, D, N):
    info = plsc.get_sparse_core_info()
    NC, NS, L = info.num_cores, info.num_subcores, info.num_lanes
    NW = NC * NS
    BW = B // NW
    CH = L
    NCH = BW // CH
    BCH = 128

    mesh = plsc.VectorSubcoreMesh(core_axis_name="c", subcore_axis_name="s")

    @functools.partial(
        pl.kernel,
        mesh=mesh,
        compiler_params=pltpu.CompilerParams(
            needs_layout_passes=False,
        ),
        out_type=jax.ShapeDtypeStruct((B,), jnp.float32),
        scratch_types=[
            pltpu.VMEM((BW,), jnp.int32),
            pltpu.VMEM((BW,), jnp.int32),
            pltpu.VMEM((2, CH, 8, D), jnp.float32),
            pltpu.VMEM((2, CH, 8, D), jnp.float32),
            pltpu.VMEM((BW,), jnp.float32),
            pltpu.VMEM((BW,), jnp.float32),
            pltpu.VMEM((BW,), jnp.float32),
            pltpu.SemaphoreType.DMA,
            pltpu.SemaphoreType.DMA,
            pltpu.SemaphoreType.DMA,
        ],
    )
    def k(item_hbm, ctx_hbm, emb2_hbm, bias_hbm, out_hbm,
          iidx, cidx, ibuf, cbuf, ibv, cbv, ov, sem0, sem1, bsem):
        wid = lax.axis_index("s") * NC + lax.axis_index("c")
        base = wid * BW
        pltpu.sync_copy(item_hbm.at[pl.ds(base, BW)], iidx)
        pltpu.sync_copy(ctx_hbm.at[pl.ds(base, BW)], cidx)

        emb_hbm = emb2_hbm.reshape(N // 8, 8, D)

        bias_copies = []
        for j in range(BW // BCH):
            sl = pl.ds(j * BCH, BCH)
            bias_copies.append(
                pltpu.async_copy(bias_hbm.at[iidx.at[sl]], ibv.at[sl], bsem))
            bias_copies.append(
                pltpu.async_copy(bias_hbm.at[cidx.at[sl]], cbv.at[sl], bsem))

        def issue_chunk(row0, p, sem):
            iv = iidx[pl.ds(row0, CH)]
            cv = cidx[pl.ds(row0, CH)]
            for l in range(CH):
                pltpu.async_copy(
                    emb_hbm.at[lax.shift_right_logical(iv[l], 3)],
                    ibuf.at[p, l], sem)
                pltpu.async_copy(
                    emb_hbm.at[lax.shift_right_logical(cv[l], 3)],
                    cbuf.at[p, l], sem)

        def wait_chunk(p, sem):
            for l in range(CH):
                pltpu.make_async_copy(emb_hbm.at[0], ibuf.at[p, l], sem).wait()
                pltpu.make_async_copy(emb_hbm.at[0], cbuf.at[p, l], sem).wait()

        lane_ids = lax.iota(jnp.int32, L)

        issue_chunk(0, 0, sem0)
        for c in bias_copies:
            c.wait()

        def body(j, carry):
            row0 = j * CH
            p = j & 1

            @pl.when(j < NCH - 1)
            def _():
                for q, s in ((0, sem0), (1, sem1)):
                    @pl.when(p != q)
                    def _():
                        issue_chunk(row0 + CH, q, s)

            for q, s in ((0, sem0), (1, sem1)):
                @pl.when(p == q)
                def _():
                    wait_chunk(q, s)

            iv = iidx[pl.ds(row0, CH)]
            cv = cidx[pl.ds(row0, CH)]
            sums = jnp.zeros((L,), jnp.float32)
            for r in range(L):
                isub = iv[r] & 7
                csub = cv[r] & 7
                acc = (ibuf[p, r, isub, pl.ds(0, L)]
                       * cbuf[p, r, csub, pl.ds(0, L)])
                for c in range(1, D // L):
                    acc = acc + (ibuf[p, r, isub, pl.ds(c * L, L)]
                                 * cbuf[p, r, csub, pl.ds(c * L, L)])
                sums = jnp.where(lane_ids == r, jnp.sum(acc), sums)
            sl = pl.ds(row0, L)
            ov[sl] = sums + ibv[sl] + cbv[sl]
            return carry

        lax.fori_loop(0, NCH, body, 0)
        pltpu.sync_copy(ov, out_hbm.at[pl.ds(base, BW)])

    return k


def kernel(item_ids, context_ids, emb_table, bias_table):
    B = item_ids.shape[0]
    N, D = emb_table.shape
    bias_flat = bias_table.reshape(-1)
    k = _make_sc_kernel(B, D, N)
    return k(item_ids.astype(jnp.int32), context_ids.astype(jnp.int32),
             emb_table, bias_flat)


# speedup vs baseline: 1.4138x; 1.0145x over previous
import functools

import jax
import jax.numpy as jnp
from jax import lax
from jax.experimental import pallas as pl
from jax.experimental.pallas import tpu as pltpu
from jax.experimental.pallas import tpu_sc as plsc


def _make_sc_kernel(B, D, N):
    info = plsc.get_sparse_core_info()
    NC, NS, L = info.num_cores, info.num_subcores, info.num_lanes
    NW = NC * NS
    BW = B // NW
    CH = L
    NCH = BW // CH
    BCH = 128

    mesh = plsc.VectorSubcoreMesh(core_axis_name="c", subcore_axis_name="s")

    @functools.partial(
        pl.kernel,
        mesh=mesh,
        compiler_params=pltpu.CompilerParams(
            needs_layout_passes=False,
        ),
        out_type=jax.ShapeDtypeStruct((B,), jnp.float32),
        scratch_types=[
            pltpu.VMEM((BW,), jnp.int32),
            pltpu.VMEM((BW,), jnp.int32),
            pltpu.VMEM((3, CH, 8, D), jnp.float32),
            pltpu.VMEM((3, CH, 8, D), jnp.float32),
            pltpu.VMEM((BW,), jnp.float32),
            pltpu.VMEM((BW,), jnp.float32),
            pltpu.VMEM((BW,), jnp.float32),
            pltpu.SemaphoreType.DMA,
            pltpu.SemaphoreType.DMA,
            pltpu.SemaphoreType.DMA,
            pltpu.SemaphoreType.DMA,
        ],
    )
    def k(item_hbm, ctx_hbm, emb2_hbm, bias_hbm, out_hbm,
          iidx, cidx, ibuf, cbuf, ibv, cbv, ov, sem0, sem1, sem2, bsem):
        wid = lax.axis_index("s") * NC + lax.axis_index("c")
        base = wid * BW
        pltpu.sync_copy(item_hbm.at[pl.ds(base, BW)], iidx)
        pltpu.sync_copy(ctx_hbm.at[pl.ds(base, BW)], cidx)

        emb_hbm = emb2_hbm.reshape(N // 8, 8, D)

        bias_copies = []
        for j in range(BW // BCH):
            sl = pl.ds(j * BCH, BCH)
            bias_copies.append(
                pltpu.async_copy(bias_hbm.at[iidx.at[sl]], ibv.at[sl], bsem))
            bias_copies.append(
                pltpu.async_copy(bias_hbm.at[cidx.at[sl]], cbv.at[sl], bsem))

        def issue_chunk(row0, p, sem):
            iv = iidx[pl.ds(row0, CH)]
            cv = cidx[pl.ds(row0, CH)]
            for l in range(CH):
                pltpu.async_copy(
                    emb_hbm.at[lax.shift_right_logical(iv[l], 3)],
                    ibuf.at[p, l], sem)
                pltpu.async_copy(
                    emb_hbm.at[lax.shift_right_logical(cv[l], 3)],
                    cbuf.at[p, l], sem)

        def wait_chunk(p, sem):
            for l in range(CH):
                pltpu.make_async_copy(emb_hbm.at[0], ibuf.at[p, l], sem).wait()
                pltpu.make_async_copy(emb_hbm.at[0], cbuf.at[p, l], sem).wait()

        lane_ids = lax.iota(jnp.int32, L)

        issue_chunk(0, 0, sem0)
        issue_chunk(CH, 1, sem1)
        for c in bias_copies:
            c.wait()

        def body(j, carry):
            row0 = j * CH
            p = j % 3

            @pl.when(j < NCH - 2)
            def _():
                for q, s in ((0, sem0), (1, sem1), (2, sem2)):
                    @pl.when((j + 2) % 3 == q)
                    def _():
                        issue_chunk(row0 + 2 * CH, q, s)

            for q, s in ((0, sem0), (1, sem1), (2, sem2)):
                @pl.when(p == q)
                def _():
                    wait_chunk(q, s)

            iv = iidx[pl.ds(row0, CH)]
            cv = cidx[pl.ds(row0, CH)]
            sums = jnp.zeros((L,), jnp.float32)
            for r in range(L):
                isub = iv[r] & 7
                csub = cv[r] & 7
                acc = (ibuf[p, r, isub, pl.ds(0, L)]
                       * cbuf[p, r, csub, pl.ds(0, L)])
                for c in range(1, D // L):
                    acc = acc + (ibuf[p, r, isub, pl.ds(c * L, L)]
                                 * cbuf[p, r, csub, pl.ds(c * L, L)])
                sums = jnp.where(lane_ids == r, jnp.sum(acc), sums)
            sl = pl.ds(row0, L)
            ov[sl] = sums + ibv[sl] + cbv[sl]
            return carry

        lax.fori_loop(0, NCH, body, 0)
        pltpu.sync_copy(ov, out_hbm.at[pl.ds(base, BW)])

    return k


def kernel(item_ids, context_ids, emb_table, bias_table):
    B = item_ids.shape[0]
    N, D = emb_table.shape
    bias_flat = bias_table.reshape(-1)
    k = _make_sc_kernel(B, D, N)
    return k(item_ids.astype(jnp.int32), context_ids.astype(jnp.int32),
             emb_table, bias_flat)


# speedup vs baseline: 2.2015x; 1.5571x over previous
import functools

import jax
import jax.numpy as jnp
from jax import lax
from jax.experimental import pallas as pl
from jax.experimental.pallas import tpu as pltpu
from jax.experimental.pallas import tpu_sc as plsc


def _make_sc_kernel(B, D, N):
    info = plsc.get_sparse_core_info()
    NC, NS, L = info.num_cores, info.num_subcores, info.num_lanes
    NW = NC * NS
    BW = B // NW
    CH = L
    NCH = BW // CH
    BCH = 128

    mesh = plsc.VectorSubcoreMesh(core_axis_name="c", subcore_axis_name="s")

    @functools.partial(
        pl.kernel,
        mesh=mesh,
        compiler_params=pltpu.CompilerParams(
            needs_layout_passes=False,
        ),
        out_type=jax.ShapeDtypeStruct((B,), jnp.float32),
        scratch_types=[
            pltpu.VMEM((BW,), jnp.int32),
            pltpu.VMEM((BW,), jnp.int32),
            pltpu.VMEM((3, CH, 8, D), jnp.float32),
            pltpu.VMEM((3, CH, 8, D), jnp.float32),
            pltpu.VMEM((BW,), jnp.float32),
            pltpu.VMEM((BW,), jnp.float32),
            pltpu.VMEM((BW,), jnp.float32),
            pltpu.SemaphoreType.DMA,
            pltpu.SemaphoreType.DMA,
            pltpu.SemaphoreType.DMA,
            pltpu.SemaphoreType.DMA,
        ],
    )
    def k(item_hbm, ctx_hbm, emb2_hbm, bias_hbm, out_hbm,
          iidx, cidx, ibuf, cbuf, ibv, cbv, ov, sem0, sem1, sem2, bsem):
        wid = lax.axis_index("s") * NC + lax.axis_index("c")
        base = wid * BW
        pltpu.sync_copy(item_hbm.at[pl.ds(base, BW)], iidx)
        pltpu.sync_copy(ctx_hbm.at[pl.ds(base, BW)], cidx)

        emb_hbm = emb2_hbm

        bias_copies = []
        for j in range(BW // BCH):
            sl = pl.ds(j * BCH, BCH)
            bias_copies.append(
                pltpu.async_copy(bias_hbm.at[iidx.at[sl]], ibv.at[sl], bsem))
            bias_copies.append(
                pltpu.async_copy(bias_hbm.at[cidx.at[sl]], cbv.at[sl], bsem))

        def issue_chunk(row0, p, sem):
            iv = iidx[pl.ds(row0, CH)]
            cv = cidx[pl.ds(row0, CH)]
            for l in range(CH):
                pltpu.async_copy(
                    emb_hbm.at[lax.shift_right_logical(iv[l], 3)],
                    ibuf.at[p, l], sem)
                pltpu.async_copy(
                    emb_hbm.at[lax.shift_right_logical(cv[l], 3)],
                    cbuf.at[p, l], sem)

        def wait_chunk(p, sem):
            for l in range(CH):
                pltpu.make_async_copy(emb_hbm.at[0], ibuf.at[p, l], sem).wait()
                pltpu.make_async_copy(emb_hbm.at[0], cbuf.at[p, l], sem).wait()

        lane_ids = lax.iota(jnp.int32, L)

        issue_chunk(0, 0, sem0)
        issue_chunk(CH, 1, sem1)
        for c in bias_copies:
            c.wait()

        def body(j, carry):
            row0 = j * CH
            p = j % 3

            @pl.when(j < NCH - 2)
            def _():
                for q, s in ((0, sem0), (1, sem1), (2, sem2)):
                    @pl.when((j + 2) % 3 == q)
                    def _():
                        issue_chunk(row0 + 2 * CH, q, s)

            for q, s in ((0, sem0), (1, sem1), (2, sem2)):
                @pl.when(p == q)
                def _():
                    wait_chunk(q, s)

            iv = iidx[pl.ds(row0, CH)]
            cv = cidx[pl.ds(row0, CH)]
            sums = jnp.zeros((L,), jnp.float32)
            for r in range(L):
                isub = iv[r] & 7
                csub = cv[r] & 7
                acc = (ibuf[p, r, isub, pl.ds(0, L)]
                       * cbuf[p, r, csub, pl.ds(0, L)])
                for c in range(1, D // L):
                    acc = acc + (ibuf[p, r, isub, pl.ds(c * L, L)]
                                 * cbuf[p, r, csub, pl.ds(c * L, L)])
                sums = jnp.where(lane_ids == r, jnp.sum(acc), sums)
            sl = pl.ds(row0, L)
            ov[sl] = sums + ibv[sl] + cbv[sl]
            return carry

        lax.fori_loop(0, NCH, body, 0)
        pltpu.sync_copy(ov, out_hbm.at[pl.ds(base, BW)])

    return k


def kernel(item_ids, context_ids, emb_table, bias_table):
    B = item_ids.shape[0]
    N, D = emb_table.shape
    bias_flat = bias_table.reshape(-1)
    emb3 = emb_table.reshape(N // 8, 8, D)
    k = _make_sc_kernel(B, D, N)
    return k(item_ids.astype(jnp.int32), context_ids.astype(jnp.int32),
             emb3, bias_flat)


# speedup vs baseline: 2.2018x; 1.0001x over previous
import functools

import jax
import jax.numpy as jnp
from jax import lax
from jax.experimental import pallas as pl
from jax.experimental.pallas import tpu as pltpu
from jax.experimental.pallas import tpu_sc as plsc


def _make_sc_kernel(B, D):
    info = plsc.get_sparse_core_info()
    NC, NS, L = info.num_cores, info.num_subcores, info.num_lanes
    NW = NC * NS
    BW = B // NW
    CH = L
    NCH = BW // CH
    BCH = 128

    mesh = plsc.VectorSubcoreMesh(core_axis_name="c", subcore_axis_name="s")

    @functools.partial(
        pl.kernel,
        mesh=mesh,
        compiler_params=pltpu.CompilerParams(
            needs_layout_passes=False,
        ),
        out_type=jax.ShapeDtypeStruct((B,), jnp.float32),
        scratch_types=[
            pltpu.VMEM((BW,), jnp.int32),
            pltpu.VMEM((BW,), jnp.int32),
            pltpu.VMEM((3, CH, 8, D), jnp.float32),
            pltpu.VMEM((3, CH, 8, D), jnp.float32),
            pltpu.VMEM((BW,), jnp.float32),
            pltpu.VMEM((BW,), jnp.float32),
            pltpu.VMEM((BW,), jnp.float32),
            pltpu.SemaphoreType.DMA,
            pltpu.SemaphoreType.DMA,
            pltpu.SemaphoreType.DMA,
            pltpu.SemaphoreType.DMA,
        ],
    )
    def k(item_hbm, ctx_hbm, emb_hbm, bias_hbm, out_hbm,
          iidx, cidx, ibuf, cbuf, ibv, cbv, ov, sem0, sem1, sem2, bsem):
        wid = lax.axis_index("s") * NC + lax.axis_index("c")
        base = wid * BW
        pltpu.sync_copy(item_hbm.at[pl.ds(base, BW)], iidx)
        pltpu.sync_copy(ctx_hbm.at[pl.ds(base, BW)], cidx)


        bias_copies = []
        for j in range(BW // BCH):
            sl = pl.ds(j * BCH, BCH)
            bias_copies.append(
                pltpu.async_copy(bias_hbm.at[iidx.at[sl]], ibv.at[sl], bsem))
            bias_copies.append(
                pltpu.async_copy(bias_hbm.at[cidx.at[sl]], cbv.at[sl], bsem))

        def issue_chunk(row0, p, sem):
            iv = iidx[pl.ds(row0, CH)]
            cv = cidx[pl.ds(row0, CH)]
            for l in range(CH):
                pltpu.async_copy(
                    emb_hbm.at[lax.shift_right_logical(iv[l], 3)],
                    ibuf.at[p, l], sem)
                pltpu.async_copy(
                    emb_hbm.at[lax.shift_right_logical(cv[l], 3)],
                    cbuf.at[p, l], sem)

        def wait_chunk(p, sem):
            for l in range(CH):
                pltpu.make_async_copy(emb_hbm.at[0], ibuf.at[p, l], sem).wait()
                pltpu.make_async_copy(emb_hbm.at[0], cbuf.at[p, l], sem).wait()

        lane_ids = lax.iota(jnp.int32, L)

        issue_chunk(0, 0, sem0)
        issue_chunk(CH, 1, sem1)
        for c in bias_copies:
            c.wait()

        def body(j, carry):
            row0 = j * CH
            p = j % 3

            @pl.when(j < NCH - 2)
            def _():
                for q, s in ((0, sem0), (1, sem1), (2, sem2)):
                    @pl.when((j + 2) % 3 == q)
                    def _():
                        issue_chunk(row0 + 2 * CH, q, s)

            for q, s in ((0, sem0), (1, sem1), (2, sem2)):
                @pl.when(p == q)
                def _():
                    wait_chunk(q, s)

            iv = iidx[pl.ds(row0, CH)]
            cv = cidx[pl.ds(row0, CH)]
            sums = jnp.zeros((L,), jnp.float32)
            for r in range(L):
                isub = iv[r] & 7
                csub = cv[r] & 7
                acc = (ibuf[p, r, isub, pl.ds(0, L)]
                       * cbuf[p, r, csub, pl.ds(0, L)])
                for c in range(1, D // L):
                    acc = acc + (ibuf[p, r, isub, pl.ds(c * L, L)]
                                 * cbuf[p, r, csub, pl.ds(c * L, L)])
                sums = jnp.where(lane_ids == r, jnp.sum(acc), sums)
            sl = pl.ds(row0, L)
            ov[sl] = sums + ibv[sl] + cbv[sl]
            return carry

        lax.fori_loop(0, NCH, body, 0)
        pltpu.sync_copy(ov, out_hbm.at[pl.ds(base, BW)])

    return k


def kernel(item_ids, context_ids, emb_table, bias_table):
    B = item_ids.shape[0]
    N, D = emb_table.shape
    bias_flat = bias_table.reshape(-1)
    emb3 = emb_table.reshape(N // 8, 8, D)
    k = _make_sc_kernel(B, D)
    return k(item_ids.astype(jnp.int32), context_ids.astype(jnp.int32),
             emb3, bias_flat)
